# baseline (speedup 1.0000x reference)
import functools

import jax
import jax.numpy as jnp
from jax import lax
from jax.experimental import pallas as pl
from jax.experimental.pallas import tpu as pltpu
from jax.experimental.pallas import tpu_sc as plsc

NC = 2
NS = 16
LANES = 16
N_HEAD = 10


def _geometry(n_nodes: int):
    per = -(-n_nodes // NS)
    stripe = -(-per // LANES) * LANES
    return stripe, NS * stripe


@functools.lru_cache(maxsize=None)
def _make_spmv3(n_nodes: int, n_edges: int):
    stripe, npad = _geometry(n_nodes)
    assert n_edges % (NS * LANES) == 0
    epw = n_edges // NS
    groups = epw // LANES

    mesh = plsc.VectorSubcoreMesh(core_axis_name="c", subcore_axis_name="s",
                                  num_cores=NC, num_subcores=NS)

    @functools.partial(
        pl.kernel,
        out_type=jax.ShapeDtypeStruct((3 * npad,), jnp.float32),
        mesh=mesh,
        compiler_params=pltpu.CompilerParams(needs_layout_passes=False),
        scratch_types=[
            pltpu.VMEM((epw,), jnp.int32),
            pltpu.VMEM((npad,), jnp.float32),
            pltpu.VMEM((npad,), jnp.float32),
            pltpu.VMEM((NS * stripe,), jnp.float32),
            pltpu.VMEM((stripe,), jnp.float32),
            pltpu.VMEM_SHARED((NS * npad,), jnp.float32),
            pltpu.VMEM_SHARED((npad,), jnp.float32),
            pltpu.SemaphoreType.DMA,
            pltpu.SemaphoreType.DMA,
        ],
    )
    def spmv3(edges_hbm, out_hbm,
              pckb, vin_v, acc_v, sbig, sacc, slots, result, sem, sem2):
        c = lax.axis_index("c")
        s = lax.axis_index("s")
        base = s * epw
        zeros16 = jnp.zeros((LANES,), jnp.float32)
        soff = s * stripe

        cp_edges = pltpu.make_async_copy(
            edges_hbm.at[pl.ds(base, epw)], pckb, sem)
        cp_edges.start()

        @plsc.parallel_loop(0, npad // LANES, unroll=8)
        def _(j):
            acc_v[pl.ds(j * LANES, LANES)] = zeros16

        cp_edges.wait()

        mask14 = jnp.full((LANES,), 0x3FFF, jnp.int32)
        w_head = jnp.full((LANES,), 1.0 / N_HEAD, jnp.float32)
        zero_w = jnp.zeros((LANES,), jnp.float32)

        for r in range(3):
            if r == 0:
                @plsc.parallel_loop(0, groups, unroll=4)
                def _(g):
                    pv = pckb[pl.ds(g * LANES, LANES)]
                    d = lax.bitwise_and(pv, mask14)
                    sv = lax.shift_right_logical(pv, 14)
                    w = jnp.where(d < N_HEAD, w_head, zero_w)
                    plsc.addupdate_scatter(acc_v, [sv], w)
            else:
                @plsc.parallel_loop(0, groups, unroll=4)
                def _(g):
                    pv = pckb[pl.ds(g * LANES, LANES)]
                    d = lax.bitwise_and(pv, mask14)
                    sv = lax.shift_right_logical(pv, 14)
                    w = plsc.load_gather(vin_v, [d])
                    plsc.addupdate_scatter(acc_v, [sv], w)

            pltpu.sync_copy(acc_v, slots.at[pl.ds(s * npad, npad)])
            plsc.subcore_barrier()
            copies = []
            for k in range(NS):
                cp = pltpu.make_async_copy(
                    slots.at[pl.ds(k * npad + soff, stripe)],
                    sbig.at[pl.ds(k * stripe, stripe)], sem)
                cp.start()
                copies.append(cp)
            for cp in copies:
                cp.wait()

            @plsc.parallel_loop(0, stripe // LANES, unroll=2)
            def _(j):
                tot = sbig[pl.ds(j * LANES, LANES)]
                for k in range(1, NS):
                    tot = tot + sbig[pl.ds(k * stripe + j * LANES, LANES)]
                sacc[pl.ds(j * LANES, LANES)] = tot

            @pl.when(c == 0)
            def _():
                pltpu.sync_copy(sacc, out_hbm.at[pl.ds(r * npad + soff, stripe)])

            if r < 2:
                pltpu.sync_copy(sacc, result.at[pl.ds(soff, stripe)])
                plsc.subcore_barrier()
                cp_vin = pltpu.make_async_copy(result, vin_v, sem2)
                cp_vin.start()

                @plsc.parallel_loop(0, npad // LANES, unroll=8)
                def _(j):
                    acc_v[pl.ds(j * LANES, LANES)] = zeros16

                cp_vin.wait()

    return spmv3, npad, stripe


@functools.lru_cache(maxsize=None)
def _make_pack(n_edges: int):
    def body(e_ref, o_ref):
        o_ref[...] = (e_ref[0, :] << 14) | e_ref[1, :]

    return pl.pallas_call(
        body,
        out_shape=jax.ShapeDtypeStruct((n_edges,), jnp.int32),
    )


@functools.lru_cache(maxsize=None)
def _make_combine(n_nodes: int, d_in: int, d_out: int, npad: int):
    chunk = 2048
    n_chunks = -(-n_nodes // chunk)
    assert n_chunks * chunk <= npad

    def body(x_ref, p_ref, w1r, w1o, b1r, w2r, w2o, b2r, w3r, w3o, b3r,
             o_ref, s_acc, c_acc):
        i = pl.program_id(0)

        iot = lax.broadcasted_iota(jnp.int32, (1, chunk), 1) + i * chunk
        v0 = jnp.where(iot < N_HEAD, jnp.float32(1.0 / N_HEAD), jnp.float32(0.0))
        v1 = p_ref[pl.ds(0 * npad + i * chunk, chunk)].reshape(1, chunk)
        v2 = p_ref[pl.ds(1 * npad + i * chunk, chunk)].reshape(1, chunk)
        v3 = p_ref[pl.ds(2 * npad + i * chunk, chunk)].reshape(1, chunk)
        V = jnp.concatenate([v0, v1, v2, v3], axis=0)
        S = lax.dot_general(V, x_ref[...], (((1,), (0,)), ((), ())),
                            preferred_element_type=jnp.float32)
        cs = jnp.sum(V, axis=1, keepdims=True)

        @pl.when(i == 0)
        def _():
            s_acc[...] = jnp.zeros_like(s_acc)
            c_acc[...] = jnp.zeros_like(c_acc)

        s_acc[...] += S
        c_acc[...] += cs

        @pl.when(i == n_chunks - 1)
        def _():
            S_t = s_acc[...]
            s0, s1, s2, s3 = S_t[0:1], S_t[1:2], S_t[2:3], S_t[3:4]
            c1, c2 = c_acc[1:2, 0:1], c_acc[2:3, 0:1]
            b1, b2, b3 = b1r[...], b2r[...], b3r[...]

            def mm(a, wref):
                return lax.dot_general(a, wref[...], (((1,), (0,)), ((), ())),
                                       preferred_element_type=jnp.float32)

            r2 = mm(s3, w1r) + c2 * b1 + mm(s2, w1o)
            r1 = mm(s2, w1r) + c1 * b1 + mm(s1, w1o)
            r0 = mm(s1, w1r) + b1 + mm(s0, w1o)
            t1 = mm(r2, w2r) + c1 * b2 + mm(r1, w2o)
            t0 = mm(r1, w2r) + b2 + mm(r0, w2o)
            o_ref[...] = mm(t1, w3r) + b3 + mm(t0, w3o)

    whole = lambda s: pl.BlockSpec(s, lambda i: tuple(0 for _ in s))
    return pl.pallas_call(
        body,
        grid=(n_chunks,),
        in_specs=[
            pl.BlockSpec((chunk, d_in), lambda i: (i, 0)),
            whole((3 * npad,)),
            whole((d_in, d_out)), whole((d_in, d_out)), whole((1, d_out)),
            whole((d_out, d_out)), whole((d_out, d_out)), whole((1, d_out)),
            whole((d_out, d_out)), whole((d_out, d_out)), whole((1, d_out)),
        ],
        out_specs=whole((1, d_out)),
        out_shape=jax.ShapeDtypeStruct((1, d_out), jnp.float32),
        scratch_shapes=[
            pltpu.VMEM((4, d_in), jnp.float32),
            pltpu.VMEM((4, 1), jnp.float32),
        ],
    )


def kernel(x, edge_index, W1_rel, W1_root, b1, W2_rel, W2_root, b2,
           W3_rel, W3_root, b3):
    n, d_in = x.shape
    d_out = W1_rel.shape[1]
    n_edges = edge_index.shape[1]
    edges = _make_pack(n_edges)(edge_index.astype(jnp.int32))

    stripe, npad = _geometry(n)
    spmv3, _, _ = _make_spmv3(n, n_edges)
    P = spmv3(edges)

    combine = _make_combine(n, d_in, d_out, npad)
    return combine(x, P,
                   W1_rel, W1_root, b1.reshape(1, d_out),
                   W2_rel, W2_root, b2.reshape(1, d_out),
                   W3_rel, W3_root, b3.reshape(1, d_out))


# speedup vs baseline: 1.0360x; 1.0360x over previous
import functools

import jax
import jax.numpy as jnp
from jax import lax
from jax.experimental import pallas as pl
from jax.experimental.pallas import tpu as pltpu
from jax.experimental.pallas import tpu_sc as plsc

NC = 2
NS = 16
LANES = 16
N_HEAD = 10


def _geometry(n_nodes: int):
    per = -(-n_nodes // NS)
    stripe = -(-per // LANES) * LANES
    return stripe, NS * stripe


@functools.lru_cache(maxsize=None)
def _make_spmv3(n_nodes: int, n_edges: int):
    stripe, npad = _geometry(n_nodes)
    assert n_edges % (NS * LANES) == 0
    epw = n_edges // NS
    groups = epw // LANES

    mesh = plsc.VectorSubcoreMesh(core_axis_name="c", subcore_axis_name="s",
                                  num_cores=NC, num_subcores=NS)

    @functools.partial(
        pl.kernel,
        out_type=jax.ShapeDtypeStruct((3 * npad,), jnp.float32),
        mesh=mesh,
        compiler_params=pltpu.CompilerParams(needs_layout_passes=False),
        scratch_types=[
            pltpu.VMEM((epw,), jnp.int32),
            pltpu.VMEM((npad,), jnp.float32),
            pltpu.VMEM((npad,), jnp.float32),
            pltpu.VMEM((NS * stripe,), jnp.float32),
            pltpu.VMEM((stripe,), jnp.float32),
            pltpu.VMEM_SHARED((NS * npad,), jnp.float32),
            pltpu.VMEM_SHARED((npad,), jnp.float32),
            pltpu.SemaphoreType.DMA,
            pltpu.SemaphoreType.DMA,
        ],
    )
    def spmv3(edges_hbm, out_hbm,
              pckb, vin_v, acc_v, sbig, sacc, slots, result, sem, sem2):
        c = lax.axis_index("c")
        s = lax.axis_index("s")
        base = s * epw
        zeros16 = jnp.zeros((LANES,), jnp.float32)
        soff = s * stripe

        cp_edges = pltpu.make_async_copy(
            edges_hbm.at[pl.ds(base, epw)], pckb, sem)
        cp_edges.start()

        @plsc.parallel_loop(0, npad // LANES, unroll=8)
        def _(j):
            acc_v[pl.ds(j * LANES, LANES)] = zeros16

        cp_edges.wait()

        mask14 = jnp.full((LANES,), 0x3FFF, jnp.int32)
        w_head = jnp.full((LANES,), 1.0 / N_HEAD, jnp.float32)
        zero_w = jnp.zeros((LANES,), jnp.float32)

        for r in range(3):
            if r == 0:
                @plsc.parallel_loop(0, groups, unroll=8)
                def _(g):
                    pv = pckb[pl.ds(g * LANES, LANES)]
                    d = lax.bitwise_and(pv, mask14)
                    sv = lax.shift_right_logical(pv, 14)
                    w = jnp.where(d < N_HEAD, w_head, zero_w)
                    plsc.addupdate_scatter(acc_v, [sv], w)
            else:
                @plsc.parallel_loop(0, groups, unroll=8)
                def _(g):
                    pv = pckb[pl.ds(g * LANES, LANES)]
                    d = lax.bitwise_and(pv, mask14)
                    sv = lax.shift_right_logical(pv, 14)
                    w = plsc.load_gather(vin_v, [d])
                    plsc.addupdate_scatter(acc_v, [sv], w)

            pltpu.sync_copy(acc_v, slots.at[pl.ds(s * npad, npad)])
            plsc.subcore_barrier()
            copies = []
            for k in range(NS):
                cp = pltpu.make_async_copy(
                    slots.at[pl.ds(k * npad + soff, stripe)],
                    sbig.at[pl.ds(k * stripe, stripe)], sem)
                cp.start()
                copies.append(cp)
            for cp in copies:
                cp.wait()

            @plsc.parallel_loop(0, stripe // LANES, unroll=4)
            def _(j):
                tot = sbig[pl.ds(j * LANES, LANES)]
                for k in range(1, NS):
                    tot = tot + sbig[pl.ds(k * stripe + j * LANES, LANES)]
                sacc[pl.ds(j * LANES, LANES)] = tot

            @pl.when(c == 0)
            def _():
                pltpu.sync_copy(sacc, out_hbm.at[pl.ds(r * npad + soff, stripe)])

            if r < 2:
                pltpu.sync_copy(sacc, result.at[pl.ds(soff, stripe)])
                plsc.subcore_barrier()
                cp_vin = pltpu.make_async_copy(result, vin_v, sem2)
                cp_vin.start()

                @plsc.parallel_loop(0, npad // LANES, unroll=8)
                def _(j):
                    acc_v[pl.ds(j * LANES, LANES)] = zeros16

                cp_vin.wait()

    return spmv3, npad, stripe


@functools.lru_cache(maxsize=None)
def _make_pack(n_edges: int):
    def body(e_ref, o_ref):
        o_ref[...] = (e_ref[0, :] << 14) | e_ref[1, :]

    return pl.pallas_call(
        body,
        out_shape=jax.ShapeDtypeStruct((n_edges,), jnp.int32),
    )


@functools.lru_cache(maxsize=None)
def _make_combine(n_nodes: int, d_in: int, d_out: int, npad: int):
    def body(x_ref, p_ref, w1r, w1o, b1r, w2r, w2o, b2r, w3r, w3o, b3r, o_ref):
        iot = lax.broadcasted_iota(jnp.int32, (1, n_nodes), 1)
        v0 = jnp.where(iot < N_HEAD, jnp.float32(1.0 / N_HEAD), jnp.float32(0.0))
        p = p_ref[...]
        v1 = p[0 * npad:1 * npad].reshape(1, npad)
        v2 = p[1 * npad:2 * npad].reshape(1, npad)
        v3 = p[2 * npad:3 * npad].reshape(1, npad)
        V = jnp.concatenate(
            [v0, v1[:, :n_nodes], v2[:, :n_nodes], v3[:, :n_nodes]], axis=0)
        csum = jnp.sum(V, axis=1, keepdims=True)
        S = lax.dot_general(V, x_ref[...], (((1,), (0,)), ((), ())),
                            preferred_element_type=jnp.float32)
        s0, s1, s2, s3 = S[0:1], S[1:2], S[2:3], S[3:4]
        c1, c2 = csum[1:2, 0:1], csum[2:3, 0:1]
        b1, b2, b3 = b1r[...], b2r[...], b3r[...]

        def mm(a, wref):
            return lax.dot_general(a, wref[...], (((1,), (0,)), ((), ())),
                                   preferred_element_type=jnp.float32)

        r2 = mm(s3, w1r) + c2 * b1 + mm(s2, w1o)
        r1 = mm(s2, w1r) + c1 * b1 + mm(s1, w1o)
        r0 = mm(s1, w1r) + b1 + mm(s0, w1o)
        t1 = mm(r2, w2r) + c1 * b2 + mm(r1, w2o)
        t0 = mm(r1, w2r) + b2 + mm(r0, w2o)
        o_ref[...] = mm(t1, w3r) + b3 + mm(t0, w3o)

    return pl.pallas_call(
        body,
        out_shape=jax.ShapeDtypeStruct((1, d_out), jnp.float32),
    )


def kernel(x, edge_index, W1_rel, W1_root, b1, W2_rel, W2_root, b2,
           W3_rel, W3_root, b3):
    n, d_in = x.shape
    d_out = W1_rel.shape[1]
    n_edges = edge_index.shape[1]
    edges = _make_pack(n_edges)(edge_index.astype(jnp.int32))

    stripe, npad = _geometry(n)
    spmv3, _, _ = _make_spmv3(n, n_edges)
    P = spmv3(edges)

    combine = _make_combine(n, d_in, d_out, npad)
    return combine(x, P,
                   W1_rel, W1_root, b1.reshape(1, d_out),
                   W2_rel, W2_root, b2.reshape(1, d_out),
                   W3_rel, W3_root, b3.reshape(1, d_out))


# speedup vs baseline: 1.0685x; 1.0314x over previous
import functools

import jax
import jax.numpy as jnp
from jax import lax
from jax.experimental import pallas as pl
from jax.experimental.pallas import tpu as pltpu
from jax.experimental.pallas import tpu_sc as plsc

NC = 2
NS = 16
LANES = 16
N_HEAD = 10


def _geometry(n_nodes: int):
    per = -(-n_nodes // NS)
    stripe = -(-per // LANES) * LANES
    return stripe, NS * stripe


@functools.lru_cache(maxsize=None)
def _make_spmv3(n_nodes: int, n_edges: int):
    stripe, npad = _geometry(n_nodes)
    assert n_edges % (NS * LANES) == 0
    epw = n_edges // NS
    groups = epw // LANES

    mesh = plsc.VectorSubcoreMesh(core_axis_name="c", subcore_axis_name="s",
                                  num_cores=NC, num_subcores=NS)

    @functools.partial(
        pl.kernel,
        out_type=(jax.ShapeDtypeStruct((2 * npad,), jnp.float32),
                  jax.ShapeDtypeStruct((NS * npad,), jnp.float32)),
        mesh=mesh,
        compiler_params=pltpu.CompilerParams(needs_layout_passes=False),
        scratch_types=[
            pltpu.VMEM((epw,), jnp.int32),
            pltpu.VMEM((npad,), jnp.float32),
            pltpu.VMEM((npad,), jnp.float32),
            pltpu.VMEM((NS * stripe,), jnp.float32),
            pltpu.VMEM((stripe,), jnp.float32),
            pltpu.VMEM_SHARED((NS * npad,), jnp.float32),
            pltpu.VMEM_SHARED((npad,), jnp.float32),
            pltpu.SemaphoreType.DMA,
            pltpu.SemaphoreType.DMA,
        ],
    )
    def spmv3(edges_hbm, out_hbm, out3_hbm,
              pckb, vin_v, acc_v, sbig, sacc, slots, result, sem, sem2):
        c = lax.axis_index("c")
        s = lax.axis_index("s")
        base = s * epw
        zeros16 = jnp.zeros((LANES,), jnp.float32)
        soff = s * stripe

        cp_edges = pltpu.make_async_copy(
            edges_hbm.at[pl.ds(base, epw)], pckb, sem)
        cp_edges.start()

        @plsc.parallel_loop(0, npad // LANES, unroll=8)
        def _(j):
            acc_v[pl.ds(j * LANES, LANES)] = zeros16

        cp_edges.wait()

        mask14 = jnp.full((LANES,), 0x3FFF, jnp.int32)
        w_head = jnp.full((LANES,), 1.0 / N_HEAD, jnp.float32)
        zero_w = jnp.zeros((LANES,), jnp.float32)

        for r in range(3):
            if r == 0:
                @plsc.parallel_loop(0, groups, unroll=4)
                def _(g):
                    pv = pckb[pl.ds(g * LANES, LANES)]
                    d = lax.bitwise_and(pv, mask14)
                    sv = lax.shift_right_logical(pv, 14)
                    w = jnp.where(d < N_HEAD, w_head, zero_w)
                    plsc.addupdate_scatter(acc_v, [sv], w)
            else:
                @plsc.parallel_loop(0, groups, unroll=4)
                def _(g):
                    pv = pckb[pl.ds(g * LANES, LANES)]
                    d = lax.bitwise_and(pv, mask14)
                    sv = lax.shift_right_logical(pv, 14)
                    w = plsc.load_gather(vin_v, [d])
                    plsc.addupdate_scatter(acc_v, [sv], w)

            if r == 2:
                @pl.when(c == 0)
                def _():
                    pltpu.sync_copy(acc_v, out3_hbm.at[pl.ds(s * npad, npad)])
                break

            pltpu.sync_copy(acc_v, slots.at[pl.ds(s * npad, npad)])
            plsc.subcore_barrier()
            copies = []
            for k in range(NS):
                cp = pltpu.make_async_copy(
                    slots.at[pl.ds(k * npad + soff, stripe)],
                    sbig.at[pl.ds(k * stripe, stripe)], sem)
                cp.start()
                copies.append(cp)
            for cp in copies:
                cp.wait()

            @plsc.parallel_loop(0, stripe // LANES, unroll=2)
            def _(j):
                tot = sbig[pl.ds(j * LANES, LANES)]
                for k in range(1, NS):
                    tot = tot + sbig[pl.ds(k * stripe + j * LANES, LANES)]
                sacc[pl.ds(j * LANES, LANES)] = tot

            @pl.when(c == 0)
            def _():
                pltpu.sync_copy(sacc, out_hbm.at[pl.ds(r * npad + soff, stripe)])

            if r < 2:
                pltpu.sync_copy(sacc, result.at[pl.ds(soff, stripe)])
                plsc.subcore_barrier()
                cp_vin = pltpu.make_async_copy(result, vin_v, sem2)
                cp_vin.start()

                @plsc.parallel_loop(0, npad // LANES, unroll=8)
                def _(j):
                    acc_v[pl.ds(j * LANES, LANES)] = zeros16

                cp_vin.wait()

    return spmv3, npad, stripe


@functools.lru_cache(maxsize=None)
def _make_pack(n_edges: int):
    def body(e_ref, o_ref):
        o_ref[...] = (e_ref[0, :] << 14) | e_ref[1, :]

    return pl.pallas_call(
        body,
        out_shape=jax.ShapeDtypeStruct((n_edges,), jnp.int32),
    )


@functools.lru_cache(maxsize=None)
def _make_combine(n_nodes: int, d_in: int, d_out: int, npad: int):
    def body(x_ref, p_ref, p3_ref, w1r, w1o, b1r, w2r, w2o, b2r, w3r, w3o, b3r,
             o_ref):
        iot = lax.broadcasted_iota(jnp.int32, (1, n_nodes), 1)
        v0 = jnp.where(iot < N_HEAD, jnp.float32(1.0 / N_HEAD), jnp.float32(0.0))
        p = p_ref[...]
        v1 = p[0 * npad:1 * npad].reshape(1, npad)
        v2 = p[1 * npad:2 * npad].reshape(1, npad)
        p3 = p3_ref[...]
        v3f = p3[0:npad]
        for k in range(1, NS):
            v3f = v3f + p3[k * npad:(k + 1) * npad]
        v3 = v3f.reshape(1, npad)
        V = jnp.concatenate(
            [v0, v1[:, :n_nodes], v2[:, :n_nodes], v3[:, :n_nodes]], axis=0)
        csum = jnp.sum(V, axis=1, keepdims=True)
        S = lax.dot_general(V, x_ref[...], (((1,), (0,)), ((), ())),
                            preferred_element_type=jnp.float32)
        s0, s1, s2, s3 = S[0:1], S[1:2], S[2:3], S[3:4]
        c1, c2 = csum[1:2, 0:1], csum[2:3, 0:1]
        b1, b2, b3 = b1r[...], b2r[...], b3r[...]

        def mm(a, wref):
            return lax.dot_general(a, wref[...], (((1,), (0,)), ((), ())),
                                   preferred_element_type=jnp.float32)

        r2 = mm(s3, w1r) + c2 * b1 + mm(s2, w1o)
        r1 = mm(s2, w1r) + c1 * b1 + mm(s1, w1o)
        r0 = mm(s1, w1r) + b1 + mm(s0, w1o)
        t1 = mm(r2, w2r) + c1 * b2 + mm(r1, w2o)
        t0 = mm(r1, w2r) + b2 + mm(r0, w2o)
        o_ref[...] = mm(t1, w3r) + b3 + mm(t0, w3o)

    return pl.pallas_call(
        body,
        out_shape=jax.ShapeDtypeStruct((1, d_out), jnp.float32),
    )


def kernel(x, edge_index, W1_rel, W1_root, b1, W2_rel, W2_root, b2,
           W3_rel, W3_root, b3):
    n, d_in = x.shape
    d_out = W1_rel.shape[1]
    n_edges = edge_index.shape[1]
    edges = _make_pack(n_edges)(edge_index.astype(jnp.int32))

    stripe, npad = _geometry(n)
    spmv3, _, _ = _make_spmv3(n, n_edges)
    P12, P3 = spmv3(edges)

    combine = _make_combine(n, d_in, d_out, npad)
    return combine(x, P12, P3,
                   W1_rel, W1_root, b1.reshape(1, d_out),
                   W2_rel, W2_root, b2.reshape(1, d_out),
                   W3_rel, W3_root, b3.reshape(1, d_out))
